# Initial kernel scaffold; baseline (speedup 1.0000x reference)
import jax
import jax.numpy as jnp
from jax.experimental import pallas as pl


def kernel(t, w, t_hat, w_hat):
    raise NotImplementedError("write your pallas kernel here")


# speedup vs baseline: 11.5528x; 11.5528x over previous
import functools

import jax
import jax.numpy as jnp
from jax.experimental import pallas as pl

_EPS = jnp.finfo(jnp.float32).eps


def _body(t_ref, w_ref, th_ref, wh_ref, out_ref):
    t = t_ref[...]
    w = w_ref[...]
    th = th_ref[...]
    wh = wh_ref[...]
    th64 = th[:, :64]
    cols = []
    for j in range(32):
        lo = t[:, j:j + 1]
        hi = t[:, j + 1:j + 2]
        m = (th64 >= lo) & (th64 <= hi)
        cols.append(jnp.sum(jnp.where(m, wh, 0.0), axis=1, keepdims=True))
    y0 = jnp.concatenate(cols, axis=1)
    d = jnp.maximum(w - y0, 0.0)
    out_ref[...] = d * d / (w + _EPS)


@jax.jit
def kernel(t, w, t_hat, w_hat):
    R = t.shape[0]
    B = 1024
    grid = (R // B,)
    return pl.pallas_call(
        _body,
        grid=grid,
        in_specs=[
            pl.BlockSpec((B, 33), lambda i: (i, 0)),
            pl.BlockSpec((B, 32), lambda i: (i, 0)),
            pl.BlockSpec((B, 65), lambda i: (i, 0)),
            pl.BlockSpec((B, 64), lambda i: (i, 0)),
        ],
        out_specs=pl.BlockSpec((B, 32), lambda i: (i, 0)),
        out_shape=jax.ShapeDtypeStruct((R, 32), jnp.float32),
    )(t, w, t_hat, w_hat)


# speedup vs baseline: 12.1247x; 1.0495x over previous
import functools

import jax
import jax.numpy as jnp
from jax import lax
from jax.experimental import pallas as pl
from jax.experimental.pallas import tpu as pltpu
from jax.experimental.pallas import tpu_sc as plsc

_EPS = jnp.finfo(jnp.float32).eps

_R = 262144
_NF = 32
_NP = 64
_G = 16

_NC, _NS, _L = 2, 16, 16
_NW = _NC * _NS
_ROWS_PER_W = _R // _NW
_GROUPS = _ROWS_PER_W // _G


def _sc_body(t_hbm, w_hbm, th_hbm, wh_hbm, out_hbm, tb, wb, thb, whb, outb):
    rows = lax.iota(jnp.int32, _G)
    r_t = rows * (_NF + 1)
    r_w = rows * _NF
    r_th = rows * (_NP + 1)
    r_wh = rows * _NP
    wid = lax.axis_index("s") * _NC + lax.axis_index("c")
    base = wid * _ROWS_PER_W

    def group_body(g, _):
        r0 = base + g * _G
        pltpu.sync_copy(t_hbm.at[pl.ds(r0 * (_NF + 1), _G * (_NF + 1))], tb)
        pltpu.sync_copy(w_hbm.at[pl.ds(r0 * _NF, _G * _NF)], wb)
        pltpu.sync_copy(th_hbm.at[pl.ds(r0 * (_NP + 1), _G * (_NP + 1))], thb)
        pltpu.sync_copy(wh_hbm.at[pl.ds(r0 * _NP, _G * _NP)], whb)

        zf = jnp.zeros((_G,), jnp.float32)
        zi = jnp.zeros((_G,), jnp.int32)

        def adv_mask(p, tj, strict):
            th = plsc.load_gather(thb, [r_th + jnp.minimum(p, _NP)])
            inb = p < (_NP + 1)
            return jnp.where(strict, th < tj, th <= tj) & inb

        def adv_loop(p, s, tj, strict):
            def cond(c):
                return jnp.any(c[2])

            def body(c):
                p, s, m = c
                wadd = plsc.load_gather(whb, [r_wh + jnp.minimum(p, _NP - 1)])
                s = s + jnp.where(m & (p < _NP), wadd, 0.0)
                p = p + m.astype(jnp.int32)
                return (p, s, adv_mask(p, tj, strict))

            p, s, _ = lax.while_loop(cond, body, (p, s, adv_mask(p, tj, strict)))
            return p, s

        def j_body(j, carry):
            pl_, pr_, sl, sr, prev_sl = carry
            tj = plsc.load_gather(tb, [r_t + j])
            pl_, sl = adv_loop(pl_, sl, tj, True)
            pr_, sr = adv_loop(pr_, sr, tj, False)
            jm1 = jnp.maximum(j - 1, 0)
            wv = plsc.load_gather(wb, [r_w + jm1])
            d = jnp.maximum(wv - (sr - prev_sl), 0.0)
            emit = jnp.broadcast_to(j >= 1, (_G,))
            plsc.store_scatter(outb, [r_w + jm1], d * d / (wv + _EPS),
                               mask=emit)
            return (pl_, pr_, sl, sr, sl)

        lax.fori_loop(0, _NF + 1, j_body, (zi, zi, zf, zf, zf))
        pltpu.sync_copy(outb, out_hbm.at[pl.ds(r0 * _NF, _G * _NF)])
        return 0

    lax.fori_loop(0, _GROUPS, group_body, 0)


@jax.jit
def kernel(t, w, t_hat, w_hat):
    mesh = plsc.VectorSubcoreMesh(core_axis_name="c", subcore_axis_name="s",
                                  num_cores=_NC, num_subcores=_NS)
    f32 = jnp.float32
    run = pl.kernel(
        _sc_body,
        out_type=jax.ShapeDtypeStruct((_R * _NF,), f32),
        mesh=mesh,
        scratch_types=[
            pltpu.VMEM((_G * (_NF + 1),), f32),
            pltpu.VMEM((_G * _NF,), f32),
            pltpu.VMEM((_G * (_NP + 1),), f32),
            pltpu.VMEM((_G * _NP,), f32),
            pltpu.VMEM((_G * _NF,), f32),
        ],
        compiler_params=pltpu.CompilerParams(needs_layout_passes=False),
    )
    out = run(t.reshape(-1), w.reshape(-1), t_hat.reshape(-1),
              w_hat.reshape(-1))
    return out.reshape(_R, _NF)


# speedup vs baseline: 23.1549x; 1.9097x over previous
import jax
import jax.numpy as jnp
from jax import lax
from jax.experimental import pallas as pl
from jax.experimental.pallas import tpu as pltpu
from jax.experimental.pallas import tpu_sc as plsc

_EPS = jnp.finfo(jnp.float32).eps

_R = 262144
_NF = 32
_NP = 64
_G = 16

_NC, _NS, _L = 2, 16, 16
_NW = _NC * _NS
_ROWS_PER_W = _R // _NW
_GR = 128
_NGRP = _ROWS_PER_W // _GR
_STRIPS = _GR // _G

_CT = _NF + 1
_CH = _NP + 1


def _sc_body(t_hbm, w_hbm, th_hbm, wh_hbm, out_hbm,
             tb0, wb0, thb0, whb0, ob0,
             tb1, wb1, thb1, whb1, ob1, sin0, sin1):
    bufs = ((tb0, wb0, thb0, whb0, ob0, sin0),
            (tb1, wb1, thb1, whb1, ob1, sin1))
    rows = lax.iota(jnp.int32, _G)
    wid = lax.axis_index("s") * _NC + lax.axis_index("c")
    base = wid * _ROWS_PER_W

    def in_copies(g, b):
        tb, wb, thb, whb, _, sem = bufs[b]
        r0 = base + g * _GR
        return (
            (t_hbm.at[pl.ds(r0 * _CT, _GR * _CT)], tb, sem),
            (w_hbm.at[pl.ds(r0 * _NF, _GR * _NF)], wb, sem),
            (th_hbm.at[pl.ds(r0 * _CH, _GR * _CH)], thb, sem),
            (wh_hbm.at[pl.ds(r0 * _NP, _GR * _NP)], whb, sem),
        )

    def start_in(g, b):
        for src, dst, sem in in_copies(g, b):
            pltpu.async_copy(src, dst, sem)

    def wait_in(g, b):
        for src, dst, sem in in_copies(g, b):
            pltpu.make_async_copy(src, dst, sem).wait()

    def compute_group(b):
        tb, wb, thb, whb, ob, _ = bufs[b]

        def strip_body(s, _):
            r_t = rows * _CT + s * (_G * _CT)
            r_w = rows * _NF + s * (_G * _NF)
            r_th = rows * _CH + s * (_G * _CH)
            r_wh = rows * _NP + s * (_G * _NP)

            def masks(p_l, p_r, tj):
                thl = plsc.load_gather(thb, [r_th + jnp.minimum(p_l, _NP)])
                thr = plsc.load_gather(thb, [r_th + jnp.minimum(p_r, _NP)])
                m_l = (thl < tj) & (p_l < _CH)
                m_r = (thr <= tj) & (p_r < _CH)
                return m_l, m_r

            def j_body(j, carry):
                p_l, p_r, s_l, s_r, prev_sl = carry
                tj = plsc.load_gather(tb, [r_t + j])
                m_l, m_r = masks(p_l, p_r, tj)

                def cond(c):
                    return jnp.any(c[4] | c[5])

                def body(c):
                    p_l, p_r, s_l, s_r, m_l, m_r = c
                    wl = plsc.load_gather(
                        whb, [r_wh + jnp.minimum(p_l, _NP - 1)])
                    wr = plsc.load_gather(
                        whb, [r_wh + jnp.minimum(p_r, _NP - 1)])
                    s_l = s_l + jnp.where(m_l & (p_l < _NP), wl, 0.0)
                    s_r = s_r + jnp.where(m_r & (p_r < _NP), wr, 0.0)
                    p_l = p_l + m_l.astype(jnp.int32)
                    p_r = p_r + m_r.astype(jnp.int32)
                    m_l, m_r = masks(p_l, p_r, tj)
                    return (p_l, p_r, s_l, s_r, m_l, m_r)

                p_l, p_r, s_l, s_r, _, _ = lax.while_loop(
                    cond, body, (p_l, p_r, s_l, s_r, m_l, m_r))

                jm1 = jnp.maximum(j - 1, 0)
                wv = plsc.load_gather(wb, [r_w + jm1])
                d = jnp.maximum(wv - (s_r - prev_sl), 0.0)
                emit = jnp.broadcast_to(j >= 1, (_G,))
                plsc.store_scatter(ob, [r_w + jm1], d * d / (wv + _EPS),
                                   mask=emit)
                return (p_l, p_r, s_l, s_r, s_l)

            zf = jnp.zeros((_G,), jnp.float32)
            zi = jnp.zeros((_G,), jnp.int32)
            lax.fori_loop(0, _CT, j_body, (zi, zi, zf, zf, zf))
            return 0

        lax.fori_loop(0, _STRIPS, strip_body, 0)

    start_in(0, 0)
    start_in(1, 1)

    def outer(i, _):
        g0 = 2 * i
        for b in range(2):
            g = g0 + b
            wait_in(g, b)
            compute_group(b)
            ob = bufs[b][4]
            r0 = base + g * _GR
            pltpu.sync_copy(ob, out_hbm.at[pl.ds(r0 * _NF, _GR * _NF)])

            @pl.when(g + 2 < _NGRP)
            def _():
                start_in(g + 2, b)
        return 0

    lax.fori_loop(0, _NGRP // 2, outer, 0)


@jax.jit
def kernel(t, w, t_hat, w_hat):
    mesh = plsc.VectorSubcoreMesh(core_axis_name="c", subcore_axis_name="s",
                                  num_cores=_NC, num_subcores=_NS)
    f32 = jnp.float32
    run = pl.kernel(
        _sc_body,
        out_type=jax.ShapeDtypeStruct((_R * _NF,), f32),
        mesh=mesh,
        scratch_types=[
            pltpu.VMEM((_GR * _CT,), f32),
            pltpu.VMEM((_GR * _NF,), f32),
            pltpu.VMEM((_GR * _CH,), f32),
            pltpu.VMEM((_GR * _NP,), f32),
            pltpu.VMEM((_GR * _NF,), f32),
            pltpu.VMEM((_GR * _CT,), f32),
            pltpu.VMEM((_GR * _NF,), f32),
            pltpu.VMEM((_GR * _CH,), f32),
            pltpu.VMEM((_GR * _NP,), f32),
            pltpu.VMEM((_GR * _NF,), f32),
            pltpu.SemaphoreType.DMA,
            pltpu.SemaphoreType.DMA,
        ],
        compiler_params=pltpu.CompilerParams(needs_layout_passes=False),
    )
    out = run(t.reshape(-1), w.reshape(-1), t_hat.reshape(-1),
              w_hat.reshape(-1))
    return out.reshape(_R, _NF)


# speedup vs baseline: 37.7387x; 1.6298x over previous
import jax
import jax.numpy as jnp
from jax import lax
from jax.experimental import pallas as pl
from jax.experimental.pallas import tpu as pltpu
from jax.experimental.pallas import tpu_sc as plsc

_EPS = jnp.finfo(jnp.float32).eps

_R = 262144
_NF = 32
_NP = 64
_G = 16

_NC, _NS, _L = 2, 16, 16
_NW = _NC * _NS
_ROWS_PER_W = _R // _NW
_GR = 128
_NGRP = _ROWS_PER_W // _GR
_STRIPS = _GR // _G

_CT = _NF + 1
_CH = _NP + 1


def _sc_body(t_hbm, w_hbm, th_hbm, wh_hbm, out_hbm,
             tb0, wb0, thb0, whb0, ob0,
             tb1, wb1, thb1, whb1, ob1, sin0, sin1):
    bufs = ((tb0, wb0, thb0, whb0, ob0, sin0),
            (tb1, wb1, thb1, whb1, ob1, sin1))
    rows = lax.iota(jnp.int32, _G)
    wid = lax.axis_index("s") * _NC + lax.axis_index("c")
    base = wid * _ROWS_PER_W

    def in_copies(g, b):
        tb, wb, thb, whb, _, sem = bufs[b]
        r0 = base + g * _GR
        return (
            (t_hbm.at[pl.ds(r0, _GR)], tb, sem),
            (w_hbm.at[pl.ds(r0, _GR)], wb, sem),
            (th_hbm.at[pl.ds(r0, _GR)], thb, sem),
            (wh_hbm.at[pl.ds(r0, _GR)], whb, sem),
        )

    def start_in(g, b):
        for src, dst, sem in in_copies(g, b):
            pltpu.async_copy(src, dst, sem)

    def wait_in(g, b):
        for src, dst, sem in in_copies(g, b):
            pltpu.make_async_copy(src, dst, sem).wait()

    def compute_group(b):
        tb, wb, thb, whb, ob, _ = bufs[b]

        def strip_body(s, _):
            rbase = s * _G + rows

            def fast_path():
                for r in range(_G):
                    for c in range(_NF // _L):
                        wv = wb[s * _G + r, pl.ds(c * _L, _L)]
                        d = jnp.maximum(wv, 0.0)
                        ob[s * _G + r, pl.ds(c * _L, _L)] = \
                            d * d / (wv + _EPS)

            def masks(p_l, p_r, tj):
                thl = plsc.load_gather(thb, [rbase, jnp.minimum(p_l, _NP)])
                thr = plsc.load_gather(thb, [rbase, jnp.minimum(p_r, _NP)])
                m_l = (thl < tj) & (p_l < _CH)
                m_r = (thr <= tj) & (p_r < _CH)
                return m_l, m_r

            def j_body(j, carry):
                p_l, p_r, s_l, s_r, prev_sl = carry
                tj = plsc.load_gather(tb, [rbase, jnp.full((_G,), j,
                                                           jnp.int32)])
                m_l, m_r = masks(p_l, p_r, tj)

                def cond(c):
                    return jnp.any(c[4] | c[5])

                def body(c):
                    p_l, p_r, s_l, s_r, m_l, m_r = c
                    wl = plsc.load_gather(
                        whb, [rbase, jnp.minimum(p_l, _NP - 1)])
                    wr = plsc.load_gather(
                        whb, [rbase, jnp.minimum(p_r, _NP - 1)])
                    s_l = s_l + jnp.where(m_l & (p_l < _NP), wl, 0.0)
                    s_r = s_r + jnp.where(m_r & (p_r < _NP), wr, 0.0)
                    p_l = p_l + m_l.astype(jnp.int32)
                    p_r = p_r + m_r.astype(jnp.int32)
                    m_l, m_r = masks(p_l, p_r, tj)
                    return (p_l, p_r, s_l, s_r, m_l, m_r)

                p_l, p_r, s_l, s_r, _, _ = lax.while_loop(
                    cond, body, (p_l, p_r, s_l, s_r, m_l, m_r))

                jm1 = jnp.maximum(j - 1, 0)
                col = jnp.full((_G,), jm1, jnp.int32)
                wv = plsc.load_gather(wb, [rbase, col])
                d = jnp.maximum(wv - (s_r - prev_sl), 0.0)
                emit = jnp.broadcast_to(j >= 1, (_G,))
                plsc.store_scatter(ob, [rbase, col], d * d / (wv + _EPS),
                                   mask=emit)
                return (p_l, p_r, s_l, s_r, s_l)

            def slow_path():
                zf = jnp.zeros((_G,), jnp.float32)
                zi = jnp.zeros((_G,), jnp.int32)
                lax.fori_loop(0, _CT, j_body, (zi, zi, zf, zf, zf))

            th0 = plsc.load_gather(thb, [rbase, jnp.zeros((_G,), jnp.int32)])
            tmax = plsc.load_gather(tb, [rbase, jnp.full((_G,), _NF,
                                                         jnp.int32)])
            lax.cond(jnp.all(th0 > tmax), fast_path, slow_path)
            return 0

        lax.fori_loop(0, _STRIPS, strip_body, 0)

    start_in(0, 0)
    start_in(1, 1)

    def outer(i, _):
        g0 = 2 * i
        for b in range(2):
            g = g0 + b
            wait_in(g, b)
            compute_group(b)
            ob = bufs[b][4]
            r0 = base + g * _GR
            pltpu.sync_copy(ob, out_hbm.at[pl.ds(r0, _GR)])

            @pl.when(g + 2 < _NGRP)
            def _():
                start_in(g + 2, b)
        return 0

    lax.fori_loop(0, _NGRP // 2, outer, 0)


@jax.jit
def kernel(t, w, t_hat, w_hat):
    mesh = plsc.VectorSubcoreMesh(core_axis_name="c", subcore_axis_name="s",
                                  num_cores=_NC, num_subcores=_NS)
    f32 = jnp.float32
    run = pl.kernel(
        _sc_body,
        out_type=jax.ShapeDtypeStruct((_R, _NF), f32),
        mesh=mesh,
        scratch_types=[
            pltpu.VMEM((_GR, _CT), f32),
            pltpu.VMEM((_GR, _NF), f32),
            pltpu.VMEM((_GR, _CH), f32),
            pltpu.VMEM((_GR, _NP), f32),
            pltpu.VMEM((_GR, _NF), f32),
            pltpu.VMEM((_GR, _CT), f32),
            pltpu.VMEM((_GR, _NF), f32),
            pltpu.VMEM((_GR, _CH), f32),
            pltpu.VMEM((_GR, _NP), f32),
            pltpu.VMEM((_GR, _NF), f32),
            pltpu.SemaphoreType.DMA,
            pltpu.SemaphoreType.DMA,
        ],
        compiler_params=pltpu.CompilerParams(needs_layout_passes=False,
                                             use_tc_tiling_on_sc=False),
    )
    return run(t, w, t_hat, w_hat)


# speedup vs baseline: 56.6955x; 1.5023x over previous
import jax
import jax.numpy as jnp
from jax import lax
from jax.experimental import pallas as pl
from jax.experimental.pallas import tpu as pltpu
from jax.experimental.pallas import tpu_sc as plsc

_EPS = jnp.finfo(jnp.float32).eps

_R = 262144
_NF = 32
_NP = 64
_G = 16

_NC, _NS, _L = 2, 16, 16
_NW = _NC * _NS
_ROWS_PER_W = _R // _NW
_GR = 64
_NGRP = _ROWS_PER_W // _GR
_STRIPS = _GR // _G

_CT = _NF + 1
_CH = _NP + 1


def _sc_body(t_hbm, w_hbm, th_hbm, wh_hbm, out_hbm,
             tb0, wb0, thb0, whb0, ob0,
             tb1, wb1, thb1, whb1, ob1, sin0, sin1):
    bufs = ((tb0, wb0, thb0, whb0, ob0, sin0),
            (tb1, wb1, thb1, whb1, ob1, sin1))
    rows = lax.iota(jnp.int32, _G)
    wid = lax.axis_index("s") * _NC + lax.axis_index("c")
    base = wid * _ROWS_PER_W

    def in_copies(g, b):
        tb, wb, thb, whb, _, sem = bufs[b]
        r0 = base + g * _GR
        return (
            (t_hbm.at[pl.ds(r0, _GR)], tb, sem),
            (w_hbm.at[pl.ds(r0, _GR)], wb, sem),
            (th_hbm.at[pl.ds(r0, _GR)], thb, sem),
            (wh_hbm.at[pl.ds(r0, _GR)], whb, sem),
        )

    def start_in(g, b):
        for src, dst, sem in in_copies(g, b):
            pltpu.async_copy(src, dst, sem)

    def wait_in(g, b):
        for src, dst, sem in in_copies(g, b):
            pltpu.make_async_copy(src, dst, sem).wait()

    def compute_group(b):
        tb, wb, thb, whb, ob, _ = bufs[b]

        def strip_body(s, _):
            rbase = s * _G + rows

            def fast_path():
                for r in range(_G):
                    for c in range(_NF // _L):
                        wv = wb[s * _G + r, pl.ds(c * _L, _L)]
                        d = jnp.maximum(wv, 0.0)
                        ob[s * _G + r, pl.ds(c * _L, _L)] = \
                            d * d / (wv + _EPS)

            def masks(p_l, p_r, tj):
                thl = plsc.load_gather(thb, [rbase, jnp.minimum(p_l, _NP)])
                thr = plsc.load_gather(thb, [rbase, jnp.minimum(p_r, _NP)])
                m_l = (thl < tj) & (p_l < _CH)
                m_r = (thr <= tj) & (p_r < _CH)
                return m_l, m_r

            def j_body(j, carry):
                p_l, p_r, s_l, s_r, prev_sl = carry
                tj = plsc.load_gather(tb, [rbase, jnp.full((_G,), j,
                                                           jnp.int32)])
                m_l, m_r = masks(p_l, p_r, tj)

                def cond(c):
                    return jnp.any(c[4] | c[5])

                def body(c):
                    p_l, p_r, s_l, s_r, m_l, m_r = c
                    wl = plsc.load_gather(
                        whb, [rbase, jnp.minimum(p_l, _NP - 1)])
                    wr = plsc.load_gather(
                        whb, [rbase, jnp.minimum(p_r, _NP - 1)])
                    s_l = s_l + jnp.where(m_l & (p_l < _NP), wl, 0.0)
                    s_r = s_r + jnp.where(m_r & (p_r < _NP), wr, 0.0)
                    p_l = p_l + m_l.astype(jnp.int32)
                    p_r = p_r + m_r.astype(jnp.int32)
                    m_l, m_r = masks(p_l, p_r, tj)
                    return (p_l, p_r, s_l, s_r, m_l, m_r)

                p_l, p_r, s_l, s_r, _, _ = lax.while_loop(
                    cond, body, (p_l, p_r, s_l, s_r, m_l, m_r))

                jm1 = jnp.maximum(j - 1, 0)
                col = jnp.full((_G,), jm1, jnp.int32)
                wv = plsc.load_gather(wb, [rbase, col])
                d = jnp.maximum(wv - (s_r - prev_sl), 0.0)
                emit = jnp.broadcast_to(j >= 1, (_G,))
                plsc.store_scatter(ob, [rbase, col], d * d / (wv + _EPS),
                                   mask=emit)
                return (p_l, p_r, s_l, s_r, s_l)

            def slow_path():
                zf = jnp.zeros((_G,), jnp.float32)
                zi = jnp.zeros((_G,), jnp.int32)
                lax.fori_loop(0, _CT, j_body, (zi, zi, zf, zf, zf))

            th0 = plsc.load_gather(thb, [rbase, jnp.zeros((_G,), jnp.int32)])
            tmax = plsc.load_gather(tb, [rbase, jnp.full((_G,), _NF,
                                                         jnp.int32)])
            lax.cond(jnp.all(th0 > tmax), fast_path, slow_path)
            return 0

        lax.fori_loop(0, _STRIPS, strip_body, 0)

    start_in(0, 0)
    start_in(1, 1)

    def outer(i, _):
        g0 = 2 * i
        for b in range(2):
            g = g0 + b
            wait_in(g, b)
            compute_group(b)
            ob = bufs[b][4]
            r0 = base + g * _GR
            pltpu.sync_copy(ob, out_hbm.at[pl.ds(r0, _GR)])

            @pl.when(g + 2 < _NGRP)
            def _():
                start_in(g + 2, b)
        return 0

    lax.fori_loop(0, _NGRP // 2, outer, 0)


@jax.jit
def kernel(t, w, t_hat, w_hat):
    mesh = plsc.VectorSubcoreMesh(core_axis_name="c", subcore_axis_name="s",
                                  num_cores=_NC, num_subcores=_NS)
    f32 = jnp.float32
    run = pl.kernel(
        _sc_body,
        out_type=jax.ShapeDtypeStruct((_R, _NF), f32),
        mesh=mesh,
        scratch_types=[
            pltpu.VMEM((_GR, _CT), f32),
            pltpu.VMEM((_GR, _NF), f32),
            pltpu.VMEM((_GR, _CH), f32),
            pltpu.VMEM((_GR, _NP), f32),
            pltpu.VMEM((_GR, _NF), f32),
            pltpu.VMEM((_GR, _CT), f32),
            pltpu.VMEM((_GR, _NF), f32),
            pltpu.VMEM((_GR, _CH), f32),
            pltpu.VMEM((_GR, _NP), f32),
            pltpu.VMEM((_GR, _NF), f32),
            pltpu.SemaphoreType.DMA,
            pltpu.SemaphoreType.DMA,
        ],
        compiler_params=pltpu.CompilerParams(needs_layout_passes=False),
    )
    return run(t, w, t_hat, w_hat)


# speedup vs baseline: 341.6010x; 6.0252x over previous
import jax
import jax.numpy as jnp
from jax import lax
from jax.experimental import pallas as pl
from jax.experimental.pallas import tpu as pltpu
from jax.experimental.pallas import tpu_sc as plsc

_EPS = jnp.finfo(jnp.float32).eps

_R = 262144
_NF = 32
_NP = 64
_G = 16

_NC, _NS, _L = 2, 16, 16
_NW = _NC * _NS
_ROWS_PER_W = _R // _NW
_GR = 128
_NGRP = _ROWS_PER_W // _GR
_STRIPS = _GR // _G

_CT = _NF + 1
_CH = _NP + 1


def _sc_body(t_hbm, w_hbm, th_hbm, wh_hbm, out_hbm,
             tb0, wb0, thb0, whb0, ob0,
             tb1, wb1, thb1, whb1, ob1, sin0, sin1):
    bufs = ((tb0, wb0, thb0, whb0, ob0, sin0),
            (tb1, wb1, thb1, whb1, ob1, sin1))
    rows = lax.iota(jnp.int32, _G)
    wid = lax.axis_index("s") * _NC + lax.axis_index("c")
    base = wid * _ROWS_PER_W

    def in_copies(g, b):
        tb, wb, thb, whb, _, sem = bufs[b]
        r0 = base + g * _GR
        return (
            (t_hbm.at[:, pl.ds(r0, _GR)], tb, sem),
            (w_hbm.at[:, pl.ds(r0, _GR)], wb, sem),
            (th_hbm.at[:, pl.ds(r0, _GR)], thb, sem),
            (wh_hbm.at[:, pl.ds(r0, _GR)], whb, sem),
        )

    def start_in(g, b):
        for src, dst, sem in in_copies(g, b):
            pltpu.async_copy(src, dst, sem)

    def wait_in(g, b):
        for src, dst, sem in in_copies(g, b):
            pltpu.make_async_copy(src, dst, sem).wait()

    def compute_group(b):
        tb, wb, thb, whb, ob, _ = bufs[b]

        def strip_body(s, _):
            c0 = s * _G
            scol = rows + c0

            def fast_path():
                for c in range(_NF):
                    wv = wb[c, pl.ds(c0, _G)]
                    d = jnp.maximum(wv, 0.0)
                    ob[c, pl.ds(c0, _G)] = d * d / (wv + _EPS)

            def masks(p_l, p_r, tj):
                thl = plsc.load_gather(thb, [jnp.minimum(p_l, _NP), scol])
                thr = plsc.load_gather(thb, [jnp.minimum(p_r, _NP), scol])
                m_l = (thl < tj) & (p_l < _CH)
                m_r = (thr <= tj) & (p_r < _CH)
                return m_l, m_r

            def advance(carry, tj):
                m_l, m_r = masks(carry[0], carry[1], tj)

                def cond(c):
                    return jnp.any(c[4] | c[5])

                def body(c):
                    p_l, p_r, s_l, s_r, m_l, m_r = c
                    wl = plsc.load_gather(
                        whb, [jnp.minimum(p_l, _NP - 1), scol])
                    wr = plsc.load_gather(
                        whb, [jnp.minimum(p_r, _NP - 1), scol])
                    s_l = s_l + jnp.where(m_l & (p_l < _NP), wl, 0.0)
                    s_r = s_r + jnp.where(m_r & (p_r < _NP), wr, 0.0)
                    p_l = p_l + m_l.astype(jnp.int32)
                    p_r = p_r + m_r.astype(jnp.int32)
                    m_l, m_r = masks(p_l, p_r, tj)
                    return (p_l, p_r, s_l, s_r, m_l, m_r)

                c = lax.while_loop(
                    cond, body, (carry[0], carry[1], carry[2], carry[3],
                                 m_l, m_r))
                return c[0], c[1], c[2], c[3]

            def j_body(j, carry):
                p_l, p_r, s_l, s_r, prev_sl = carry
                tj = tb[j, pl.ds(c0, _G)]
                p_l, p_r, s_l, s_r = advance((p_l, p_r, s_l, s_r), tj)
                jm1 = j - 1
                wv = wb[jm1, pl.ds(c0, _G)]
                d = jnp.maximum(wv - (s_r - prev_sl), 0.0)
                ob[jm1, pl.ds(c0, _G)] = d * d / (wv + _EPS)
                return (p_l, p_r, s_l, s_r, s_l)

            def slow_path():
                zf = jnp.zeros((_G,), jnp.float32)
                zi = jnp.zeros((_G,), jnp.int32)
                t0 = tb[0, pl.ds(c0, _G)]
                p_l, p_r, s_l, s_r = advance((zi, zi, zf, zf), t0)
                lax.fori_loop(1, _CT, j_body, (p_l, p_r, s_l, s_r, s_l))

            th0 = thb[0, pl.ds(c0, _G)]
            tmax = tb[_NF, pl.ds(c0, _G)]
            lax.cond(jnp.all(th0 > tmax), fast_path, slow_path)
            return 0

        lax.fori_loop(0, _STRIPS, strip_body, 0)

    start_in(0, 0)
    start_in(1, 1)

    def outer(i, _):
        g0 = 2 * i
        for b in range(2):
            g = g0 + b
            wait_in(g, b)
            compute_group(b)
            ob = bufs[b][4]
            r0 = base + g * _GR
            pltpu.sync_copy(ob, out_hbm.at[:, pl.ds(r0, _GR)])

            @pl.when(g + 2 < _NGRP)
            def _():
                start_in(g + 2, b)
        return 0

    lax.fori_loop(0, _NGRP // 2, outer, 0)


@jax.jit
def kernel(t, w, t_hat, w_hat):
    mesh = plsc.VectorSubcoreMesh(core_axis_name="c", subcore_axis_name="s",
                                  num_cores=_NC, num_subcores=_NS)
    f32 = jnp.float32
    run = pl.kernel(
        _sc_body,
        out_type=jax.ShapeDtypeStruct((_NF, _R), f32),
        mesh=mesh,
        scratch_types=[
            pltpu.VMEM((_CT, _GR), f32),
            pltpu.VMEM((_NF, _GR), f32),
            pltpu.VMEM((_CH, _GR), f32),
            pltpu.VMEM((_NP, _GR), f32),
            pltpu.VMEM((_NF, _GR), f32),
            pltpu.VMEM((_CT, _GR), f32),
            pltpu.VMEM((_NF, _GR), f32),
            pltpu.VMEM((_CH, _GR), f32),
            pltpu.VMEM((_NP, _GR), f32),
            pltpu.VMEM((_NF, _GR), f32),
            pltpu.SemaphoreType.DMA,
            pltpu.SemaphoreType.DMA,
        ],
        compiler_params=pltpu.CompilerParams(needs_layout_passes=False),
    )
    out_t = run(t.T, w.T, t_hat.T, w_hat.T)
    return out_t.T
